# Initial kernel scaffold; baseline (speedup 1.0000x reference)
import jax
import jax.numpy as jnp
from jax.experimental import pallas as pl


def kernel(x_seq, edge_index, edge_weight, params):
    raise NotImplementedError("write your pallas kernel here")


# speedup vs baseline: 2.3729x; 2.3729x over previous
import functools

import jax
import jax.numpy as jnp
from jax import lax
from jax.experimental import pallas as pl
from jax.experimental.pallas import tpu as pltpu
from jax.experimental.pallas import tpu_sc as plsc

B0, T0, N0, F0 = 2, 8, 1000, 8
G0 = B0 * T0
HID = 64
HEADS = 4
HK = HEADS * HID
NP = 1024
NC, NS = 2, 16
NPW = NP // NS
EE = 16000 + N0
EW = 1088
EP = EW * NS
CH = 64
GPC = G0 // NC
KB = 2560
F32 = jnp.float32


def _proj_body(x_ref, w_ref, b_ref, o_ref):
    o_ref[...] = (
        jnp.dot(x_ref[...], w_ref[...], preferred_element_type=F32) + b_ref[...]
    )


def _gat_dense_body(x_ref, w_ref, asrc_ref, adst_ref, h_ref, as_ref, ad_ref,
                    c_ref):
    h = jnp.dot(x_ref[...], w_ref[...], preferred_element_type=F32)
    h_ref[...] = h
    a_s = jnp.dot(h, asrc_ref[...], preferred_element_type=F32)
    a_d = jnp.dot(h, adst_ref[...], preferred_element_type=F32)
    as_ref[...] = a_s
    ad_ref[...] = a_d
    ms = jnp.max(a_s.reshape(G0, NP, HEADS), axis=1)
    md = jnp.max(a_d.reshape(G0, NP, HEADS), axis=1)
    s = ms + md
    c_ref[...] = jnp.where(s > 0, s, 0.2 * s)


def _gi_body(x_ref, w_ref, o_ref):
    @pl.when(pl.program_id(0) == 0)
    def _():
        o_ref[...] = jnp.zeros_like(o_ref)

    o_ref[...] += lax.dot_general(
        x_ref[...], w_ref[...], (((1,), (1,)), ((), ())),
        preferred_element_type=F32)


def _gru_body(gir_ref, giz_ref, gin_ref, whr_ref, whz_ref, whn_ref,
              bi_ref, bh_ref, wfc_ref, bfc_ref, o_ref):
    h = jnp.zeros((B0, HID), F32)
    for t in range(T0):
        hr = lax.dot_general(h, whr_ref[...], (((1,), (1,)), ((), ())),
                             preferred_element_type=F32) + bh_ref[0:1]
        hz = lax.dot_general(h, whz_ref[...], (((1,), (1,)), ((), ())),
                             preferred_element_type=F32) + bh_ref[1:2]
        hn = lax.dot_general(h, whn_ref[...], (((1,), (1,)), ((), ())),
                             preferred_element_type=F32) + bh_ref[2:3]
        r = jax.nn.sigmoid(gir_ref[t] + bi_ref[0:1] + hr)
        z = jax.nn.sigmoid(giz_ref[t] + bi_ref[1:2] + hz)
        n = jnp.tanh(gin_ref[t] + bi_ref[2:3] + r * hn)
        h = (1.0 - z) * n + z * h
    o_ref[...] = (
        jnp.dot(h, wfc_ref[...], preferred_element_type=F32) + bfc_ref[...]
    )


def _iota16():
    return lax.iota(jnp.int32, 16)


def _splat(v):
    return jnp.full((16,), v, jnp.int32)


def _edge_body(h_hbm, asd_hbm, cmax_hbm, src_hbm, dst_hbm,
               bias_hbm, xout_hbm,
               den_sp, agg_sp,
               srcv, dstv, arows, drows, exloc, exstage, hrows, contrib,
               obuf, cbuf, bvec, sem):
    c = lax.axis_index("c")
    s = lax.axis_index("s")
    ebase = s * EW
    row0 = s * NPW

    pltpu.sync_copy(cmax_hbm.at[c], cbuf)
    pltpu.sync_copy(bias_hbm, bvec)

    def zrow(i, _):
        for p in range(HID // 16):
            obuf[i, pl.ds(p * 16, 16)] = jnp.zeros((16,), F32)
        return 0
    lax.fori_loop(0, NPW, zrow, 0)
    pltpu.sync_copy(obuf.at[:, pl.ds(0, 32)], den_sp.at[pl.ds(row0, NPW)])
    for gi in range(GPC):
        pltpu.sync_copy(obuf, agg_sp.at[gi].at[pl.ds(row0, NPW)])
    plsc.subcore_barrier()

    def chunk1(ci, _):
        e0 = ebase + ci * CH
        pltpu.sync_copy(src_hbm.at[pl.ds(e0, CH)], srcv)
        pltpu.sync_copy(dst_hbm.at[pl.ds(e0, CH)], dstv)
        pltpu.async_copy(asd_hbm.at[c].at[srcv], arows, sem).wait()
        pltpu.async_copy(asd_hbm.at[c].at[dstv], drows, sem).wait()

        def grp(jj, _):
            lanes = _iota16()
            ridx = jj * 16 + lanes
            for q in range(32):
                qi = _splat(q)
                av = plsc.load_gather(arows, [ridx, qi])
                dv = plsc.load_gather(drows, [ridx, _splat(32 + q)])
                cq = plsc.load_gather(cbuf, [qi])
                v = av + dv
                v = jnp.where(v > 0.0, v, 0.2 * v)
                ex = jnp.exp(v - cq)
                plsc.store_scatter(exstage, [ridx, qi], ex)
            return 0
        lax.fori_loop(0, CH // 16, grp, 0)
        pltpu.sync_copy(exstage, den_sp.at[dstv], add=True)
        return 0
    lax.fori_loop(0, EW // CH, chunk1, 0)
    plsc.subcore_barrier()

    def chunk2(ci, _):
        e0 = ebase + ci * CH
        pltpu.sync_copy(src_hbm.at[pl.ds(e0, CH)], srcv)
        pltpu.sync_copy(dst_hbm.at[pl.ds(e0, CH)], dstv)
        pltpu.async_copy(asd_hbm.at[c].at[srcv], arows, sem).wait()
        pltpu.async_copy(asd_hbm.at[c].at[dstv], drows, sem).wait()
        pltpu.async_copy(den_sp.at[dstv], exstage, sem).wait()

        def grp(jj, _):
            lanes = _iota16()
            ridx = jj * 16 + lanes
            for q in range(32):
                qi = _splat(q)
                av = plsc.load_gather(arows, [ridx, qi])
                dv = plsc.load_gather(drows, [ridx, _splat(32 + q)])
                cq = plsc.load_gather(cbuf, [qi])
                dn = plsc.load_gather(exstage, [ridx, qi])
                v = av + dv
                v = jnp.where(v > 0.0, v, 0.2 * v)
                ex = jnp.exp(v - cq)
                plsc.store_scatter(exloc, [ci * CH + ridx, qi],
                                   0.25 * ex / (dn + 1e-16))
            return 0
        lax.fori_loop(0, CH // 16, grp, 0)
        return 0
    lax.fori_loop(0, EW // CH, chunk2, 0)

    def graph_loop(gi, _):
        g = c * GPC + gi

        def chunkb(ci, _):
            e0 = ebase + ci * CH
            pltpu.sync_copy(src_hbm.at[pl.ds(e0, CH)], srcv)
            pltpu.sync_copy(dst_hbm.at[pl.ds(e0, CH)], dstv)
            pltpu.async_copy(h_hbm.at[g].at[srcv], hrows, sem).wait()

            def grp(jj, _):
                lanes = _iota16()
                ridx = jj * 16 + lanes
                evec = ci * CH + ridx
                alph = []
                for k in range(HEADS):
                    col = _splat(gi * HEADS + k)
                    alph.append(plsc.load_gather(exloc, [evec, col]))
                for q in range(HID):
                    v = None
                    for k in range(HEADS):
                        hv = plsc.load_gather(hrows,
                                              [ridx, _splat(k * HID + q)])
                        v = hv * alph[k] if v is None else v + hv * alph[k]
                    plsc.store_scatter(contrib, [ridx, _splat(q)], v)
                return 0
            lax.fori_loop(0, CH // 16, grp, 0)
            pltpu.sync_copy(contrib, agg_sp.at[gi].at[dstv], add=True)
            return 0
        lax.fori_loop(0, EW // CH, chunkb, 0)
        return 0
    lax.fori_loop(0, GPC, graph_loop, 0)
    plsc.subcore_barrier()

    def flush_loop(gi, _):
        g = c * GPC + gi
        pltpu.sync_copy(agg_sp.at[gi].at[pl.ds(row0, NPW)], obuf)

        def frow(i, _):
            for p in range(HID // 16):
                sl = pl.ds(p * 16, 16)
                v = obuf[i, sl] + bvec[sl]
                obuf[i, sl] = jnp.where(v > 0.0, v, jnp.exp(v) - 1.0)
            return 0
        lax.fori_loop(0, NPW, frow, 0)
        pltpu.sync_copy(obuf, xout_hbm.at[g].at[pl.ds(row0, NPW)])
        return 0
    lax.fori_loop(0, GPC, flush_loop, 0)


_EDGE_KERNEL = pl.kernel(
    _edge_body,
    out_type=jax.ShapeDtypeStruct((G0, NP, HID), F32),
    mesh=plsc.VectorSubcoreMesh(core_axis_name="c", subcore_axis_name="s"),
    compiler_params=pltpu.CompilerParams(needs_layout_passes=False,
                                         use_tc_tiling_on_sc=False),
    scratch_types=[
        pltpu.VMEM_SHARED((NP, 32), F32),
        pltpu.VMEM_SHARED((GPC, NP, HID), F32),
        pltpu.VMEM((CH,), jnp.int32),
        pltpu.VMEM((CH,), jnp.int32),
        pltpu.VMEM((CH, 64), F32),
        pltpu.VMEM((CH, 64), F32),
        pltpu.VMEM((EW, 32), F32),
        pltpu.VMEM((CH, 32), F32),
        pltpu.VMEM((CH, HK), F32),
        pltpu.VMEM((CH, HID), F32),
        pltpu.VMEM((NPW, HID), F32),
        pltpu.VMEM((32,), F32),
        pltpu.VMEM((HID,), F32),
        pltpu.SemaphoreType.DMA,
    ],
)


_PROJ = pl.pallas_call(
    _proj_body,
    out_shape=jax.ShapeDtypeStruct((G0 * NP, HID), F32),
)

_GAT_DENSE = pl.pallas_call(
    _gat_dense_body,
    out_shape=(
        jax.ShapeDtypeStruct((G0 * NP, HK), F32),
        jax.ShapeDtypeStruct((G0 * NP, HEADS), F32),
        jax.ShapeDtypeStruct((G0 * NP, HEADS), F32),
        jax.ShapeDtypeStruct((G0, HEADS), F32),
    ),
)

_GI = pl.pallas_call(
    _gi_body,
    grid=(64000 // KB,),
    in_specs=[
        pl.BlockSpec((G0, KB), lambda k: (0, k)),
        pl.BlockSpec((192, KB), lambda k: (0, k)),
    ],
    out_specs=pl.BlockSpec((G0, 192), lambda k: (0, 0)),
    out_shape=jax.ShapeDtypeStruct((G0, 192), F32),
)

_GRU = pl.pallas_call(
    _gru_body,
    out_shape=jax.ShapeDtypeStruct((B0, N0), F32),
)


def kernel(x_seq, edge_index, edge_weight, params):
    del edge_weight

    loops = jnp.arange(N0, dtype=edge_index.dtype)
    pad = jnp.full((EP - EE,), N0, edge_index.dtype)
    src = jnp.concatenate([edge_index[0], loops, pad]).astype(jnp.int32)
    dst = jnp.concatenate([edge_index[1], loops, pad]).astype(jnp.int32)

    x0 = x_seq.reshape(G0, N0, F0)
    x0 = jnp.pad(x0, ((0, 0), (0, NP - N0), (0, 0)))
    x = _PROJ(x0.reshape(G0 * NP, F0), params['W_in'],
              params['b_in'].reshape(1, HID))

    eye = jnp.eye(HEADS, dtype=F32)
    for p in params['gat']:
        asrc_bd = (eye[:, None, :] * p['a_src'][:, :, None]).reshape(HK, HEADS)
        adst_bd = (eye[:, None, :] * p['a_dst'][:, :, None]).reshape(HK, HEADS)
        h, a_s, a_d, cmax = _GAT_DENSE(x, p['W'], asrc_bd, adst_bd)
        h = h.reshape(G0, NP, HK)
        a_s = a_s.reshape(NC, GPC, NP, HEADS).transpose(0, 2, 1, 3)
        a_s = a_s.reshape(NC, NP, GPC * HEADS)
        a_d = a_d.reshape(NC, GPC, NP, HEADS).transpose(0, 2, 1, 3)
        a_d = a_d.reshape(NC, NP, GPC * HEADS)
        asd = jnp.concatenate([a_s, a_d], axis=2)
        cmax = cmax.reshape(NC, GPC * HEADS)
        xn = _EDGE_KERNEL(h, asd, cmax, src, dst, p['b'])
        x = xn.reshape(G0 * NP, HID)

    xf = x.reshape(G0, NP, HID)[:, :N0, :].reshape(G0, N0 * HID)
    gi = _GI(xf, params['W_ih'])

    gi3 = gi.reshape(B0, T0, 3, HID).transpose(1, 0, 2, 3)
    gir, giz, gin = gi3[:, :, 0], gi3[:, :, 1], gi3[:, :, 2]
    whr = params['W_hh'][0:HID]
    whz = params['W_hh'][HID:2 * HID]
    whn = params['W_hh'][2 * HID:3 * HID]
    bi = params['b_ih'].reshape(3, HID)
    bh = params['b_hh'].reshape(3, HID)
    out = _GRU(gir, giz, gin, whr, whz, whn, bi, bh,
               params['W_fc'], params['b_fc'].reshape(1, N0))
    return out


# speedup vs baseline: 6.8447x; 2.8845x over previous
import functools

import jax
import jax.numpy as jnp
from jax import lax
from jax.experimental import pallas as pl
from jax.experimental.pallas import tpu as pltpu
from jax.experimental.pallas import tpu_sc as plsc

B0, T0, N0, F0 = 2, 8, 1000, 8
G0 = B0 * T0
HID = 64
HEADS = 4
HK = HEADS * HID
NP = 1024
NC, NS = 2, 16
NPW = NP // NS
EE = 16000 + N0
EW = 1088
EP = EW * NS
CH = 64
GPC = G0 // NC
KB = 2560
F32 = jnp.float32


def _proj_body(x_ref, w_ref, b_ref, o_ref):
    o_ref[...] = (
        jnp.dot(x_ref[...], w_ref[...], preferred_element_type=F32) + b_ref[...]
    )


def _gat_dense_body(x_ref, w_ref, asrc_ref, adst_ref, h_ref, as_ref, ad_ref,
                    c_ref):
    h = jnp.dot(x_ref[...], w_ref[...], preferred_element_type=F32)
    h_ref[...] = h
    a_s = jnp.dot(h, asrc_ref[...], preferred_element_type=F32)
    a_d = jnp.dot(h, adst_ref[...], preferred_element_type=F32)
    as_ref[...] = a_s
    ad_ref[...] = a_d
    ms = jnp.max(a_s.reshape(G0, NP, HEADS), axis=1)
    md = jnp.max(a_d.reshape(G0, NP, HEADS), axis=1)
    s = ms + md
    c_ref[...] = jnp.where(s > 0, s, 0.2 * s)


def _gi_body(x_ref, w_ref, o_ref):
    @pl.when(pl.program_id(0) == 0)
    def _():
        o_ref[...] = jnp.zeros_like(o_ref)

    o_ref[...] += lax.dot_general(
        x_ref[...], w_ref[...], (((1,), (1,)), ((), ())),
        preferred_element_type=F32)


def _gru_body(gir_ref, giz_ref, gin_ref, whr_ref, whz_ref, whn_ref,
              bi_ref, bh_ref, wfc_ref, bfc_ref, o_ref):
    h = jnp.zeros((B0, HID), F32)
    for t in range(T0):
        hr = lax.dot_general(h, whr_ref[...], (((1,), (1,)), ((), ())),
                             preferred_element_type=F32) + bh_ref[0:1]
        hz = lax.dot_general(h, whz_ref[...], (((1,), (1,)), ((), ())),
                             preferred_element_type=F32) + bh_ref[1:2]
        hn = lax.dot_general(h, whn_ref[...], (((1,), (1,)), ((), ())),
                             preferred_element_type=F32) + bh_ref[2:3]
        r = jax.nn.sigmoid(gir_ref[t] + bi_ref[0:1] + hr)
        z = jax.nn.sigmoid(giz_ref[t] + bi_ref[1:2] + hz)
        n = jnp.tanh(gin_ref[t] + bi_ref[2:3] + r * hn)
        h = (1.0 - z) * n + z * h
    o_ref[...] = (
        jnp.dot(h, wfc_ref[...], preferred_element_type=F32) + bfc_ref[...]
    )


def _iota16():
    return lax.iota(jnp.int32, 16)


def _splat(v):
    return jnp.full((16,), v, jnp.int32)


def _edge_body(h_hbm, asd_hbm, cmax_hbm, src_hbm, dst_hbm,
               bias_hbm, xout_hbm,
               den_sp, agg_sp,
               srcm, dstm, arows, drows, exloc, exstage, hrows, contrib,
               obuf, cbuf, bvec, sem):
    c = lax.axis_index("c")
    s = lax.axis_index("s")
    ebase = s * EW
    row0 = s * NPW

    pltpu.sync_copy(cmax_hbm.at[c], cbuf)
    pltpu.sync_copy(bias_hbm, bvec)

    def zrow(i, _):
        for p in range(HID // 16):
            obuf[i, pl.ds(p * 16, 16)] = jnp.zeros((16,), F32)
        return 0
    lax.fori_loop(0, NPW, zrow, 0)
    pltpu.sync_copy(obuf.at[:, pl.ds(0, 32)], den_sp.at[pl.ds(row0, NPW)])
    for gi in range(GPC):
        pltpu.sync_copy(obuf, agg_sp.at[gi].at[pl.ds(row0, NPW)])
    plsc.subcore_barrier()

    def idxrow(ci, _):
        pltpu.sync_copy(src_hbm.at[pl.ds(ebase + ci * CH, CH)], srcm.at[ci])
        pltpu.sync_copy(dst_hbm.at[pl.ds(ebase + ci * CH, CH)], dstm.at[ci])
        return 0
    lax.fori_loop(0, EW // CH, idxrow, 0)

    cq0 = cbuf[pl.ds(0, 16)]
    cq1 = cbuf[pl.ds(16, 16)]

    def chunk1(ci, _):
        pltpu.async_copy(asd_hbm.at[c].at[srcm.at[ci]], arows, sem).wait()
        pltpu.async_copy(asd_hbm.at[c].at[dstm.at[ci]], drows, sem).wait()

        def edge(j, _):
            for p in range(2):
                v = arows[j, pl.ds(p * 16, 16)] + drows[j, pl.ds(32 + p * 16, 16)]
                v = jnp.where(v > 0.0, v, 0.2 * v)
                ex = jnp.exp(v - (cq0 if p == 0 else cq1))
                exloc[ci * CH + j, pl.ds(p * 16, 16)] = ex
            return 0
        lax.fori_loop(0, CH, edge, 0)
        pltpu.sync_copy(exloc.at[pl.ds(ci * CH, CH)], den_sp.at[dstm.at[ci]],
                        add=True)
        return 0
    lax.fori_loop(0, EW // CH, chunk1, 0)
    plsc.subcore_barrier()

    def chunk2(ci, _):
        pltpu.async_copy(den_sp.at[dstm.at[ci]], exstage, sem).wait()

        def edge(j, _):
            for p in range(2):
                sl = pl.ds(p * 16, 16)
                ex = exloc[ci * CH + j, sl]
                dn = exstage[j, sl]
                exloc[ci * CH + j, sl] = 0.25 * ex / (dn + 1e-16)
            return 0
        lax.fori_loop(0, CH, edge, 0)
        return 0
    lax.fori_loop(0, EW // CH, chunk2, 0)

    def graph_loop(gi, _):
        g = c * GPC + gi

        def chunkb(ci, _):
            pltpu.async_copy(h_hbm.at[g].at[srcm.at[ci]], hrows, sem).wait()

            def edge(j, _):
                el = ci * CH + j
                al = []
                for k in range(HEADS):
                    al.append(plsc.load_gather(
                        exloc, [_splat(el), _splat(gi * HEADS + k)]))
                for p in range(HID // 16):
                    acc = None
                    for k in range(HEADS):
                        hv = hrows[j, pl.ds(k * HID + p * 16, 16)]
                        acc = hv * al[k] if acc is None else acc + hv * al[k]
                    contrib[j, pl.ds(p * 16, 16)] = acc
                return 0
            lax.fori_loop(0, CH, edge, 0)
            pltpu.sync_copy(contrib, agg_sp.at[gi].at[dstm.at[ci]], add=True)
            return 0
        lax.fori_loop(0, EW // CH, chunkb, 0)
        return 0
    lax.fori_loop(0, GPC, graph_loop, 0)
    plsc.subcore_barrier()

    def flush_loop(gi, _):
        g = c * GPC + gi
        pltpu.sync_copy(agg_sp.at[gi].at[pl.ds(row0, NPW)], obuf)

        def frow(i, _):
            for p in range(HID // 16):
                sl = pl.ds(p * 16, 16)
                v = obuf[i, sl] + bvec[sl]
                obuf[i, sl] = jnp.where(v > 0.0, v, jnp.exp(v) - 1.0)
            return 0
        lax.fori_loop(0, NPW, frow, 0)
        pltpu.sync_copy(obuf, xout_hbm.at[g].at[pl.ds(row0, NPW)])
        return 0
    lax.fori_loop(0, GPC, flush_loop, 0)


_EDGE_KERNEL = pl.kernel(
    _edge_body,
    out_type=jax.ShapeDtypeStruct((G0, NP, HID), F32),
    mesh=plsc.VectorSubcoreMesh(core_axis_name="c", subcore_axis_name="s"),
    compiler_params=pltpu.CompilerParams(needs_layout_passes=False,
                                         use_tc_tiling_on_sc=False),
    scratch_types=[
        pltpu.VMEM_SHARED((NP, 32), F32),
        pltpu.VMEM_SHARED((GPC, NP, HID), F32),
        pltpu.VMEM((EW // CH, CH), jnp.int32),
        pltpu.VMEM((EW // CH, CH), jnp.int32),
        pltpu.VMEM((CH, 64), F32),
        pltpu.VMEM((CH, 64), F32),
        pltpu.VMEM((EW, 32), F32),
        pltpu.VMEM((CH, 32), F32),
        pltpu.VMEM((CH, HK), F32),
        pltpu.VMEM((CH, HID), F32),
        pltpu.VMEM((NPW, HID), F32),
        pltpu.VMEM((32,), F32),
        pltpu.VMEM((HID,), F32),
        pltpu.SemaphoreType.DMA,
    ],
)


_PROJ = pl.pallas_call(
    _proj_body,
    out_shape=jax.ShapeDtypeStruct((G0 * NP, HID), F32),
)

_GAT_DENSE = pl.pallas_call(
    _gat_dense_body,
    out_shape=(
        jax.ShapeDtypeStruct((G0 * NP, HK), F32),
        jax.ShapeDtypeStruct((G0 * NP, HEADS), F32),
        jax.ShapeDtypeStruct((G0 * NP, HEADS), F32),
        jax.ShapeDtypeStruct((G0, HEADS), F32),
    ),
)

_GI = pl.pallas_call(
    _gi_body,
    grid=(64000 // KB,),
    in_specs=[
        pl.BlockSpec((G0, KB), lambda k: (0, k)),
        pl.BlockSpec((192, KB), lambda k: (0, k)),
    ],
    out_specs=pl.BlockSpec((G0, 192), lambda k: (0, 0)),
    out_shape=jax.ShapeDtypeStruct((G0, 192), F32),
)

_GRU = pl.pallas_call(
    _gru_body,
    out_shape=jax.ShapeDtypeStruct((B0, N0), F32),
)


def kernel(x_seq, edge_index, edge_weight, params):
    del edge_weight

    loops = jnp.arange(N0, dtype=edge_index.dtype)
    pad = jnp.full((EP - EE,), N0, edge_index.dtype)
    src = jnp.concatenate([edge_index[0], loops, pad]).astype(jnp.int32)
    dst = jnp.concatenate([edge_index[1], loops, pad]).astype(jnp.int32)

    x0 = x_seq.reshape(G0, N0, F0)
    x0 = jnp.pad(x0, ((0, 0), (0, NP - N0), (0, 0)))
    x = _PROJ(x0.reshape(G0 * NP, F0), params['W_in'],
              params['b_in'].reshape(1, HID))

    eye = jnp.eye(HEADS, dtype=F32)
    for p in params['gat']:
        asrc_bd = (eye[:, None, :] * p['a_src'][:, :, None]).reshape(HK, HEADS)
        adst_bd = (eye[:, None, :] * p['a_dst'][:, :, None]).reshape(HK, HEADS)
        h, a_s, a_d, cmax = _GAT_DENSE(x, p['W'], asrc_bd, adst_bd)
        h = h.reshape(G0, NP, HK)
        a_s = a_s.reshape(NC, GPC, NP, HEADS).transpose(0, 2, 1, 3)
        a_s = a_s.reshape(NC, NP, GPC * HEADS)
        a_d = a_d.reshape(NC, GPC, NP, HEADS).transpose(0, 2, 1, 3)
        a_d = a_d.reshape(NC, NP, GPC * HEADS)
        asd = jnp.concatenate([a_s, a_d], axis=2)
        cmax = cmax.reshape(NC, GPC * HEADS)
        xn = _EDGE_KERNEL(h, asd, cmax, src, dst, p['b'])
        x = xn.reshape(G0 * NP, HID)

    xf = x.reshape(G0, NP, HID)[:, :N0, :].reshape(G0, N0 * HID)
    gi = _GI(xf, params['W_ih'])

    gi3 = gi.reshape(B0, T0, 3, HID).transpose(1, 0, 2, 3)
    gir, giz, gin = gi3[:, :, 0], gi3[:, :, 1], gi3[:, :, 2]
    whr = params['W_hh'][0:HID]
    whz = params['W_hh'][HID:2 * HID]
    whn = params['W_hh'][2 * HID:3 * HID]
    bi = params['b_ih'].reshape(3, HID)
    bh = params['b_hh'].reshape(3, HID)
    out = _GRU(gir, giz, gin, whr, whz, whn, bi, bh,
               params['W_fc'], params['b_fc'].reshape(1, N0))
    return out
